# Initial kernel scaffold; baseline (speedup 1.0000x reference)
import jax
import jax.numpy as jnp
from jax.experimental import pallas as pl


def kernel(conds, inputs, W):
    raise NotImplementedError("write your pallas kernel here")


# speedup vs baseline: 4.9765x; 4.9765x over previous
import functools

import jax
import jax.numpy as jnp
from jax import lax
from jax.experimental import pallas as pl
from jax.experimental.pallas import tpu as pltpu
from jax.experimental.pallas import tpu_sc as plsc

_N = 1000
_B = 16384
_NC = 2
_NS = 16
_NW = _NC * _NS
_L = 16
_BPW = _B // _NW
_IDX_ROWS = _BPW // 128


def _adj_body(w_ref, wadj_ref):
    w = w_ref[...]
    m = jnp.max(w, axis=1, keepdims=True)
    s = jnp.sum(jnp.exp(w - m), axis=1, keepdims=True)
    wadj_ref[...] = w - (m + jnp.log(s))


def _adjust_table(W):
    return pl.pallas_call(
        _adj_body,
        out_shape=jax.ShapeDtypeStruct((_N, _N), jnp.float32),
    )(W)


def _gather_body(conds_hbm, inputs_hbm, wflat_hbm, out_hbm,
                 conds_v, inputs_v, flat_v, vals_v, sem):
    wid = lax.axis_index("s") * _NC + lax.axis_index("c")
    base = wid * _BPW
    pltpu.sync_copy(conds_hbm.at[pl.ds(base, _BPW)], conds_v)
    pltpu.sync_copy(inputs_hbm.at[pl.ds(base, _BPW)], inputs_v)
    for j in range(_IDX_ROWS):
        for k in range(128 // _L):
            off = j * 128 + k * _L
            c = conds_v[pl.ds(off, _L)]
            i = inputs_v[pl.ds(off, _L)]
            flat_v[j, pl.ds(k * _L, _L)] = c * _N + i
    copies = [
        pltpu.async_copy(wflat_hbm.at[flat_v.at[j]],
                         vals_v.at[pl.ds(j * 128, 128)], sem)
        for j in range(_IDX_ROWS)
    ]
    for c_ in copies:
        c_.wait()
    pltpu.sync_copy(vals_v, out_hbm.at[pl.ds(base, _BPW)])


_gather_call = functools.partial(
    pl.kernel,
    out_type=jax.ShapeDtypeStruct((_B,), jnp.float32),
    mesh=plsc.VectorSubcoreMesh(core_axis_name="c", subcore_axis_name="s"),
    scratch_types=[
        pltpu.VMEM((_BPW,), jnp.int32),
        pltpu.VMEM((_BPW,), jnp.int32),
        pltpu.VMEM((_IDX_ROWS, 128), jnp.int32),
        pltpu.VMEM((_BPW,), jnp.float32),
        pltpu.SemaphoreType.DMA,
    ],
)(_gather_body)


def kernel(conds, inputs, W):
    wadj = _adjust_table(W)
    return _gather_call(conds.astype(jnp.int32), inputs.astype(jnp.int32),
                        wadj.reshape(_N * _N))
